# baseline (speedup 1.0000x reference)
import functools

import jax
import jax.numpy as jnp
from jax import lax
from jax.experimental import pallas as pl
from jax.experimental.pallas import tpu as pltpu
from jax.experimental.pallas import tpu_sc as plsc

N_NODES = 10000
D = 128
E = 160000

NSUB = 16
R = 632
NPAD = NSUB * R
DUMP = R
C = 4000
NCHUNK = E // C
G = 128


def _segmax_worker(src_hbm, dst_hbm, x_hbm, out_hbm, sub,
                   src_v, dst_v, csrc_v, cdst_v, rows_v, acc_v, sem):
    lo = sub * R
    iota = lax.iota(jnp.int32, 16)
    minf = jnp.full((16,), -jnp.inf, dtype=jnp.float32)

    def init_body(i, _):
        acc_v[i, pl.ds(0, 16)] = minf
        return 0
    def init_row(i, _):
        for j in range(D // 16):
            acc_v[i, pl.ds(j * 16, 16)] = minf
        return 0
    lax.fori_loop(0, R + 1, init_row, 0)

    def chunk_body(ch, _):
        pltpu.sync_copy(src_hbm.at[pl.ds(ch * C, C)], src_v)
        pltpu.sync_copy(dst_hbm.at[pl.ds(ch * C, C)], dst_v)

        def scan_body(g, n):
            d = dst_v[pl.ds(g * 16, 16)]
            s = src_v[pl.ds(g * 16, 16)]
            dl = d - lo
            m = (dl >= 0) & (dl < R)
            mi = m.astype(jnp.int32)
            pos = n + plsc.cumsum(mi) - 1
            plsc.store_scatter(csrc_v, [pos], s, mask=m)
            plsc.store_scatter(cdst_v, [pos], dl, mask=m)
            return n + jnp.sum(mi)
        n = lax.fori_loop(0, C // 16, scan_body, jnp.int32(0))

        for i in range(G // 16):
            csrc_v[pl.ds(n + i * 16, 16)] = jnp.zeros((16,), jnp.int32)
            cdst_v[pl.ds(n + i * 16, 16)] = jnp.full((16,), DUMP, jnp.int32)

        nb = (n + G - 1) // G

        def batch_body(b, _):
            pltpu.async_copy(x_hbm.at[csrc_v.at[pl.ds(b * G, G)]],
                             rows_v, sem).wait()

            def grp_body(g, _):
                dlv = cdst_v[pl.ds(b * G + g * 16, 16)]
                for k in range(16):
                    dl = dlv[k]
                    e = g * 16 + k
                    for j in range(D // 16):
                        cs = pl.ds(j * 16, 16)
                        acc_v[dl, cs] = jnp.maximum(acc_v[dl, cs],
                                                    rows_v[e, cs])
                return 0
            lax.fori_loop(0, G // 16, grp_body, 0)
            return 0
        lax.fori_loop(0, nb, batch_body, 0)
        return 0
    lax.fori_loop(0, NCHUNK, chunk_body, 0)

    pltpu.sync_copy(acc_v.at[pl.ds(0, R)], out_hbm.at[pl.ds(lo, R)])


def _sc_body(x_hbm, uu_src, uu_dst, ui_src, ui_dst, agg_uu, agg_ui,
             src_v, dst_v, csrc_v, cdst_v, rows_v, acc_v, sem):
    core = lax.axis_index("c")
    sub = lax.axis_index("s")

    @pl.when(core == 0)
    def _():
        _segmax_worker(uu_src, uu_dst, x_hbm, agg_uu, sub,
                       src_v, dst_v, csrc_v, cdst_v, rows_v, acc_v, sem)

    @pl.when(core == 1)
    def _():
        _segmax_worker(ui_src, ui_dst, x_hbm, agg_ui, sub,
                       src_v, dst_v, csrc_v, cdst_v, rows_v, acc_v, sem)


def _segmax_both(x_user, uu_src, uu_dst, ui_src, ui_dst):
    mesh = plsc.VectorSubcoreMesh(core_axis_name="c", subcore_axis_name="s")
    f = functools.partial(
        pl.kernel,
        mesh=mesh,
        compiler_params=pltpu.CompilerParams(needs_layout_passes=False),
        out_type=[
            jax.ShapeDtypeStruct((NPAD, D), jnp.float32),
            jax.ShapeDtypeStruct((NPAD, D), jnp.float32),
        ],
        scratch_types=[
            pltpu.VMEM((C,), jnp.int32),
            pltpu.VMEM((C,), jnp.int32),
            pltpu.VMEM((C + G,), jnp.int32),
            pltpu.VMEM((C + G,), jnp.int32),
            pltpu.VMEM((G, D), jnp.float32),
            pltpu.VMEM((R + 1, D), jnp.float32),
            pltpu.SemaphoreType.DMA,
        ],
    )(_sc_body)
    return f(x_user, uu_src, uu_dst, ui_src, ui_dst)


def _linear_body(x_ref, agg_ref, w_ref, b_ref, o_ref):
    agg = agg_ref[...]
    agg = jnp.where(jnp.isfinite(agg), agg, 0.0)
    o_ref[...] = (jnp.dot(x_ref[...] + agg, w_ref[...],
                          preferred_element_type=jnp.float32)
                  + b_ref[...])


def _gin_linear(x, agg, W, b):
    blk = 1000
    grid = (N_NODES // blk,)
    return pl.pallas_call(
        _linear_body,
        grid=grid,
        in_specs=[
            pl.BlockSpec((blk, D), lambda i: (i, 0)),
            pl.BlockSpec((blk, D), lambda i: (i, 0)),
            pl.BlockSpec((D, D), lambda i: (0, 0)),
            pl.BlockSpec((1, D), lambda i: (0, 0)),
        ],
        out_specs=pl.BlockSpec((blk, D), lambda i: (i, 0)),
        out_shape=jax.ShapeDtypeStruct((N_NODES, D), jnp.float32),
    )(x, agg, W, b.reshape(1, D))


def kernel(x_user, x_item, edge_index_uu, edge_index_ui,
           W_follows, b_follows, W_clicks, b_clicks):
    agg_uu, agg_ui = _segmax_both(
        x_user,
        edge_index_uu[0], edge_index_uu[1],
        edge_index_ui[0], edge_index_ui[1],
    )
    h_user = _gin_linear(x_user, agg_uu[:N_NODES], W_follows, b_follows)
    h_item = _gin_linear(x_item, agg_ui[:N_NODES], W_clicks, b_clicks)
    return (h_user, h_item)


# speedup vs baseline: 4.0321x; 4.0321x over previous
import functools

import jax
import jax.numpy as jnp
from jax import lax
from jax.experimental import pallas as pl
from jax.experimental.pallas import tpu as pltpu
from jax.experimental.pallas import tpu_sc as plsc

N_NODES = 10000
D = 128
E = 160000

NSUB = 16
R = 632
NPAD = NSUB * R
DUMP = R
C = 4000
NCHUNK = E // C
G = 128


def _segmax_worker(src_hbm, dst_hbm, x_hbm, out_hbm, sub,
                   src_v, dst_v, csrc_v, cdst_v, rows_v, acc_v, sem):
    lo = sub * R
    iota = lax.iota(jnp.int32, 16)
    minf = jnp.full((16,), -jnp.inf, dtype=jnp.float32)

    def init_row(i, _):
        for j in range(D // 16):
            acc_v[i, pl.ds(j * 16, 16)] = minf
        return 0
    lax.fori_loop(0, R + 1, init_row, 0)

    def batch_body(b, _):
        pltpu.async_copy(x_hbm.at[csrc_v.at[pl.ds(b * G, G)]],
                         rows_v, sem).wait()

        def grp_body(g, _):
            dlv = cdst_v[pl.ds(b * G + g * 16, 16)]
            for k in range(16):
                dl = dlv[k]
                e = g * 16 + k
                for j in range(D // 16):
                    cs = pl.ds(j * 16, 16)
                    acc_v[dl, cs] = jnp.maximum(acc_v[dl, cs],
                                                rows_v[e, cs])
            return 0
        lax.fori_loop(0, G // 16, grp_body, 0)
        return 0

    def chunk_body(ch, n):
        pltpu.sync_copy(src_hbm.at[pl.ds(ch * C, C)], src_v)
        pltpu.sync_copy(dst_hbm.at[pl.ds(ch * C, C)], dst_v)

        def scan_body(g, n):
            d = dst_v[pl.ds(g * 16, 16)]
            s = src_v[pl.ds(g * 16, 16)]
            dl = d - lo
            m = (dl >= 0) & (dl < R)
            mi = m.astype(jnp.int32)
            pos = n + plsc.cumsum(mi) - 1
            plsc.store_scatter(csrc_v, [pos], s, mask=m)
            plsc.store_scatter(cdst_v, [pos], dl, mask=m)
            return n + jnp.sum(mi)
        n = lax.fori_loop(0, C // 16, scan_body, n)

        nfull = n // G
        lax.fori_loop(0, nfull, batch_body, 0)

        for i in range(G // 16):
            sv = csrc_v[pl.ds(nfull * G + i * 16, 16)]
            dv = cdst_v[pl.ds(nfull * G + i * 16, 16)]
            csrc_v[pl.ds(i * 16, 16)] = sv
            cdst_v[pl.ds(i * 16, 16)] = dv
        return n - nfull * G

    n = lax.fori_loop(0, NCHUNK, chunk_body, jnp.int32(0))

    for i in range(G // 16):
        csrc_v[pl.ds(n + i * 16, 16)] = iota + (i * 16)
        cdst_v[pl.ds(n + i * 16, 16)] = jnp.full((16,), DUMP, jnp.int32)
    nb = (n + G - 1) // G
    lax.fori_loop(0, nb, batch_body, 0)

    pltpu.sync_copy(acc_v.at[pl.ds(0, R)], out_hbm.at[pl.ds(lo, R)])


def _sc_body(x_hbm, uu_src, uu_dst, ui_src, ui_dst, agg_uu, agg_ui,
             src_v, dst_v, csrc_v, cdst_v, rows_v, acc_v, sem):
    core = lax.axis_index("c")
    sub = lax.axis_index("s")

    @pl.when(core == 0)
    def _():
        _segmax_worker(uu_src, uu_dst, x_hbm, agg_uu, sub,
                       src_v, dst_v, csrc_v, cdst_v, rows_v, acc_v, sem)

    @pl.when(core == 1)
    def _():
        _segmax_worker(ui_src, ui_dst, x_hbm, agg_ui, sub,
                       src_v, dst_v, csrc_v, cdst_v, rows_v, acc_v, sem)


def _segmax_both(x_user, uu_src, uu_dst, ui_src, ui_dst):
    mesh = plsc.VectorSubcoreMesh(core_axis_name="c", subcore_axis_name="s")
    f = functools.partial(
        pl.kernel,
        mesh=mesh,
        compiler_params=pltpu.CompilerParams(needs_layout_passes=False),
        out_type=[
            jax.ShapeDtypeStruct((NPAD, D), jnp.float32),
            jax.ShapeDtypeStruct((NPAD, D), jnp.float32),
        ],
        scratch_types=[
            pltpu.VMEM((C,), jnp.int32),
            pltpu.VMEM((C,), jnp.int32),
            pltpu.VMEM((C + G,), jnp.int32),
            pltpu.VMEM((C + G,), jnp.int32),
            pltpu.VMEM((G, D), jnp.float32),
            pltpu.VMEM((R + 1, D), jnp.float32),
            pltpu.SemaphoreType.DMA,
        ],
    )(_sc_body)
    return f(x_user, uu_src, uu_dst, ui_src, ui_dst)


def _linear_body(x_ref, agg_ref, w_ref, b_ref, o_ref):
    agg = agg_ref[...]
    agg = jnp.where(jnp.isfinite(agg), agg, 0.0)
    o_ref[...] = (jnp.dot(x_ref[...] + agg, w_ref[...],
                          preferred_element_type=jnp.float32)
                  + b_ref[...])


def _gin_linear(x, agg, W, b):
    blk = 1000
    grid = (N_NODES // blk,)
    return pl.pallas_call(
        _linear_body,
        grid=grid,
        in_specs=[
            pl.BlockSpec((blk, D), lambda i: (i, 0)),
            pl.BlockSpec((blk, D), lambda i: (i, 0)),
            pl.BlockSpec((D, D), lambda i: (0, 0)),
            pl.BlockSpec((1, D), lambda i: (0, 0)),
        ],
        out_specs=pl.BlockSpec((blk, D), lambda i: (i, 0)),
        out_shape=jax.ShapeDtypeStruct((N_NODES, D), jnp.float32),
    )(x, agg, W, b.reshape(1, D))


def kernel(x_user, x_item, edge_index_uu, edge_index_ui,
           W_follows, b_follows, W_clicks, b_clicks):
    agg_uu, agg_ui = _segmax_both(
        x_user,
        edge_index_uu[0], edge_index_uu[1],
        edge_index_ui[0], edge_index_ui[1],
    )
    h_user = _gin_linear(x_user, agg_uu[:N_NODES], W_follows, b_follows)
    h_item = _gin_linear(x_item, agg_ui[:N_NODES], W_clicks, b_clicks)
    return (h_user, h_item)


# speedup vs baseline: 6.1578x; 1.5272x over previous
import functools

import jax
import jax.numpy as jnp
from jax import lax
from jax.experimental import pallas as pl
from jax.experimental.pallas import tpu as pltpu
from jax.experimental.pallas import tpu_sc as plsc

N_NODES = 10000
D = 128
E = 160000

NSUB = 16
R = 632
NPAD = NSUB * R
DUMP = R
C = 4000
NCHUNK = E // C
G = 128


def _segmax_worker(src_hbm, dst_hbm, x_hbm, out_hbm, sub,
                   src_v, dst_v, csrc_v, cdst_v, gidx_v, gdl_v,
                   rows_v, acc_v, sem):
    lo = sub * R
    iota = lax.iota(jnp.int32, 16)
    minf = jnp.full((16,), -jnp.inf, dtype=jnp.float32)

    def init_row(i, _):
        for j in range(D // 16):
            acc_v[i, pl.ds(j * 16, 16)] = minf
        return 0
    lax.fori_loop(0, R + 1, init_row, 0)

    def accumulate():
        def grp_body(g, _):
            dlv = gdl_v[pl.ds(g * 16, 16)]
            for k in range(16):
                dl = dlv[k]
                e = g * 16 + k
                msgs = [rows_v[e, pl.ds(j * 16, 16)]
                        for j in range(D // 16)]
                for j in range(D // 16):
                    cs = pl.ds(j * 16, 16)
                    acc_v[dl, cs] = jnp.maximum(acc_v[dl, cs], msgs[j])
            return 0
        lax.fori_loop(0, G // 16, grp_body, 0)

    def flush_now():
        pltpu.make_async_copy(x_hbm.at[gidx_v], rows_v, sem).wait()
        accumulate()

    def issue(b):
        for i in range(G // 16):
            gidx_v[pl.ds(i * 16, 16)] = csrc_v[pl.ds(b * G + i * 16, 16)]
            gdl_v[pl.ds(i * 16, 16)] = cdst_v[pl.ds(b * G + i * 16, 16)]
        pltpu.async_copy(x_hbm.at[gidx_v], rows_v, sem)

    def chunk_body(ch, carry):
        n, p = carry
        pltpu.sync_copy(src_hbm.at[pl.ds(ch * C, C)], src_v)
        pltpu.sync_copy(dst_hbm.at[pl.ds(ch * C, C)], dst_v)

        def scan_body(g, n):
            d = dst_v[pl.ds(g * 16, 16)]
            s = src_v[pl.ds(g * 16, 16)]
            dl = d - lo
            m = (dl >= 0) & (dl < R)
            mi = m.astype(jnp.int32)
            pos = n + plsc.cumsum(mi) - 1
            plsc.store_scatter(csrc_v, [pos], s, mask=m)
            plsc.store_scatter(cdst_v, [pos], dl, mask=m)
            return n + jnp.sum(mi)
        n = lax.fori_loop(0, C // 16, scan_body, n)

        nfull = n // G

        def bloop(b, p):
            @pl.when(p == 1)
            def _():
                flush_now()
            issue(b)
            return jnp.int32(1)
        p = lax.fori_loop(0, nfull, bloop, p)

        for i in range(G // 16):
            sv = csrc_v[pl.ds(nfull * G + i * 16, 16)]
            dv = cdst_v[pl.ds(nfull * G + i * 16, 16)]
            csrc_v[pl.ds(i * 16, 16)] = sv
            cdst_v[pl.ds(i * 16, 16)] = dv
        return (n - nfull * G, p)

    n, p = lax.fori_loop(0, NCHUNK, chunk_body,
                         (jnp.int32(0), jnp.int32(0)))

    @pl.when(p == 1)
    def _():
        flush_now()

    for i in range(G // 16):
        csrc_v[pl.ds(n + i * 16, 16)] = iota + (i * 16)
        cdst_v[pl.ds(n + i * 16, 16)] = jnp.full((16,), DUMP, jnp.int32)
    nb = (n + G - 1) // G

    def tail_body(b, _):
        issue(b)
        flush_now()
        return 0
    lax.fori_loop(0, nb, tail_body, 0)

    pltpu.sync_copy(acc_v.at[pl.ds(0, R)], out_hbm.at[pl.ds(lo, R)])


def _sc_body(x_hbm, uu_src, uu_dst, ui_src, ui_dst, agg_uu, agg_ui,
             src_v, dst_v, csrc_v, cdst_v, gidx_v, gdl_v,
             rows_v, acc_v, sem):
    core = lax.axis_index("c")
    sub = lax.axis_index("s")

    @pl.when(core == 0)
    def _():
        _segmax_worker(uu_src, uu_dst, x_hbm, agg_uu, sub,
                       src_v, dst_v, csrc_v, cdst_v, gidx_v, gdl_v,
                       rows_v, acc_v, sem)

    @pl.when(core == 1)
    def _():
        _segmax_worker(ui_src, ui_dst, x_hbm, agg_ui, sub,
                       src_v, dst_v, csrc_v, cdst_v, gidx_v, gdl_v,
                       rows_v, acc_v, sem)


def _segmax_both(x_user, uu_src, uu_dst, ui_src, ui_dst):
    mesh = plsc.VectorSubcoreMesh(core_axis_name="c", subcore_axis_name="s")
    f = functools.partial(
        pl.kernel,
        mesh=mesh,
        compiler_params=pltpu.CompilerParams(needs_layout_passes=False),
        out_type=[
            jax.ShapeDtypeStruct((NPAD, D), jnp.float32),
            jax.ShapeDtypeStruct((NPAD, D), jnp.float32),
        ],
        scratch_types=[
            pltpu.VMEM((C,), jnp.int32),
            pltpu.VMEM((C,), jnp.int32),
            pltpu.VMEM((C + G,), jnp.int32),
            pltpu.VMEM((C + G,), jnp.int32),
            pltpu.VMEM((G,), jnp.int32),
            pltpu.VMEM((G,), jnp.int32),
            pltpu.VMEM((G, D), jnp.float32),
            pltpu.VMEM((R + 1, D), jnp.float32),
            pltpu.SemaphoreType.DMA,
        ],
    )(_sc_body)
    return f(x_user, uu_src, uu_dst, ui_src, ui_dst)


def _linear_body(x_ref, agg_ref, w_ref, b_ref, o_ref):
    agg = agg_ref[...]
    agg = jnp.where(jnp.isfinite(agg), agg, 0.0)
    o_ref[...] = (jnp.dot(x_ref[...] + agg, w_ref[...],
                          preferred_element_type=jnp.float32)
                  + b_ref[...])


def _gin_linear(x, agg, W, b):
    blk = 1000
    grid = (N_NODES // blk,)
    return pl.pallas_call(
        _linear_body,
        grid=grid,
        in_specs=[
            pl.BlockSpec((blk, D), lambda i: (i, 0)),
            pl.BlockSpec((blk, D), lambda i: (i, 0)),
            pl.BlockSpec((D, D), lambda i: (0, 0)),
            pl.BlockSpec((1, D), lambda i: (0, 0)),
        ],
        out_specs=pl.BlockSpec((blk, D), lambda i: (i, 0)),
        out_shape=jax.ShapeDtypeStruct((N_NODES, D), jnp.float32),
    )(x, agg, W, b.reshape(1, D))


def kernel(x_user, x_item, edge_index_uu, edge_index_ui,
           W_follows, b_follows, W_clicks, b_clicks):
    agg_uu, agg_ui = _segmax_both(
        x_user,
        edge_index_uu[0], edge_index_uu[1],
        edge_index_ui[0], edge_index_ui[1],
    )
    h_user = _gin_linear(x_user, agg_uu[:N_NODES], W_follows, b_follows)
    h_item = _gin_linear(x_item, agg_ui[:N_NODES], W_clicks, b_clicks)
    return (h_user, h_item)


# speedup vs baseline: 7.1849x; 1.1668x over previous
import functools

import jax
import jax.numpy as jnp
from jax import lax
from jax.experimental import pallas as pl
from jax.experimental.pallas import tpu as pltpu
from jax.experimental.pallas import tpu_sc as plsc

N_NODES = 10000
D = 128
E = 160000

NSUB = 16
R = 632
NPAD = NSUB * R
DUMP = R
C = 4000
NCHUNK = E // C
G = 128


def _segmax_worker(src_hbm, dst_hbm, x_hbm, out_hbm, sub,
                   src_v, dst_v, src_b, dst_b, csrc_v, cdst_v,
                   gidx_v, gdl_v, rows_v, acc_v, sem, csem_a, csem_b):
    lo = sub * R
    iota = lax.iota(jnp.int32, 16)
    minf = jnp.full((16,), -jnp.inf, dtype=jnp.float32)

    def init_row(i, _):
        for j in range(D // 16):
            acc_v[i, pl.ds(j * 16, 16)] = minf
        return 0
    lax.fori_loop(0, R + 1, init_row, 0)

    def accumulate():
        def grp_body(g, _):
            dlv = gdl_v[pl.ds(g * 16, 16)]
            for k in range(16):
                dl = dlv[k]
                e = g * 16 + k
                msgs = [rows_v[e, pl.ds(j * 16, 16)]
                        for j in range(D // 16)]
                for j in range(D // 16):
                    cs = pl.ds(j * 16, 16)
                    acc_v[dl, cs] = jnp.maximum(acc_v[dl, cs], msgs[j])
            return 0
        lax.fori_loop(0, G // 16, grp_body, 0)

    def flush_now():
        pltpu.make_async_copy(x_hbm.at[gidx_v], rows_v, sem).wait()
        accumulate()

    def issue(b):
        for i in range(G // 16):
            gidx_v[pl.ds(i * 16, 16)] = csrc_v[pl.ds(b * G + i * 16, 16)]
            gdl_v[pl.ds(i * 16, 16)] = cdst_v[pl.ds(b * G + i * 16, 16)]
        pltpu.async_copy(x_hbm.at[gidx_v], rows_v, sem)

    def issue_chunk(ch, sbuf, dbuf, csem):
        pltpu.async_copy(src_hbm.at[pl.ds(ch * C, C)], sbuf, csem)
        pltpu.async_copy(dst_hbm.at[pl.ds(ch * C, C)], dbuf, csem)

    def wait_chunk(ch, sbuf, dbuf, csem):
        pltpu.make_async_copy(src_hbm.at[pl.ds(ch * C, C)], sbuf,
                              csem).wait()
        pltpu.make_async_copy(dst_hbm.at[pl.ds(ch * C, C)], dbuf,
                              csem).wait()

    def process_chunk(sbuf, dbuf, carry):
        n, p = carry

        def scan_body(g, n):
            d = dbuf[pl.ds(g * 16, 16)]
            s = sbuf[pl.ds(g * 16, 16)]
            dl = d - lo
            m = (dl >= 0) & (dl < R)
            mi = m.astype(jnp.int32)
            pos = n + plsc.cumsum(mi) - 1
            plsc.store_scatter(csrc_v, [pos], s, mask=m)
            plsc.store_scatter(cdst_v, [pos], dl, mask=m)
            return n + jnp.sum(mi)
        n = lax.fori_loop(0, C // 16, scan_body, n)

        nfull = n // G

        def bloop(b, p):
            @pl.when(p == 1)
            def _():
                flush_now()
            issue(b)
            return jnp.int32(1)
        p = lax.fori_loop(0, nfull, bloop, p)

        for i in range(G // 16):
            sv = csrc_v[pl.ds(nfull * G + i * 16, 16)]
            dv = cdst_v[pl.ds(nfull * G + i * 16, 16)]
            csrc_v[pl.ds(i * 16, 16)] = sv
            cdst_v[pl.ds(i * 16, 16)] = dv
        return (n - nfull * G, p)

    issue_chunk(0, src_v, dst_v, csem_a)

    def super_body(i, carry):
        wait_chunk(2 * i, src_v, dst_v, csem_a)
        issue_chunk(2 * i + 1, src_b, dst_b, csem_b)
        carry = process_chunk(src_v, dst_v, carry)

        @pl.when(i < NCHUNK // 2 - 1)
        def _():
            issue_chunk(2 * i + 2, src_v, dst_v, csem_a)
        wait_chunk(2 * i + 1, src_b, dst_b, csem_b)
        carry = process_chunk(src_b, dst_b, carry)
        return carry

    n, p = lax.fori_loop(0, NCHUNK // 2, super_body,
                         (jnp.int32(0), jnp.int32(0)))

    @pl.when(p == 1)
    def _():
        flush_now()

    for i in range(G // 16):
        csrc_v[pl.ds(n + i * 16, 16)] = iota + (i * 16)
        cdst_v[pl.ds(n + i * 16, 16)] = jnp.full((16,), DUMP, jnp.int32)
    nb = (n + G - 1) // G

    def tail_body(b, _):
        issue(b)
        flush_now()
        return 0
    lax.fori_loop(0, nb, tail_body, 0)

    pltpu.sync_copy(acc_v.at[pl.ds(0, R)], out_hbm.at[pl.ds(lo, R)])


def _sc_body(x_hbm, uu_src, uu_dst, ui_src, ui_dst, agg_uu, agg_ui,
             src_v, dst_v, src_b, dst_b, csrc_v, cdst_v, gidx_v, gdl_v,
             rows_v, acc_v, sem, csem_a, csem_b):
    core = lax.axis_index("c")
    sub = lax.axis_index("s")

    @pl.when(core == 0)
    def _():
        _segmax_worker(uu_src, uu_dst, x_hbm, agg_uu, sub,
                       src_v, dst_v, src_b, dst_b, csrc_v, cdst_v,
                       gidx_v, gdl_v, rows_v, acc_v, sem, csem_a, csem_b)

    @pl.when(core == 1)
    def _():
        _segmax_worker(ui_src, ui_dst, x_hbm, agg_ui, sub,
                       src_v, dst_v, src_b, dst_b, csrc_v, cdst_v,
                       gidx_v, gdl_v, rows_v, acc_v, sem, csem_a, csem_b)


def _segmax_both(x_user, uu_src, uu_dst, ui_src, ui_dst):
    mesh = plsc.VectorSubcoreMesh(core_axis_name="c", subcore_axis_name="s")
    f = functools.partial(
        pl.kernel,
        mesh=mesh,
        compiler_params=pltpu.CompilerParams(needs_layout_passes=False),
        out_type=[
            jax.ShapeDtypeStruct((NPAD, D), jnp.float32),
            jax.ShapeDtypeStruct((NPAD, D), jnp.float32),
        ],
        scratch_types=[
            pltpu.VMEM((C,), jnp.int32),
            pltpu.VMEM((C,), jnp.int32),
            pltpu.VMEM((C,), jnp.int32),
            pltpu.VMEM((C,), jnp.int32),
            pltpu.VMEM((C + G,), jnp.int32),
            pltpu.VMEM((C + G,), jnp.int32),
            pltpu.VMEM((G,), jnp.int32),
            pltpu.VMEM((G,), jnp.int32),
            pltpu.VMEM((G, D), jnp.float32),
            pltpu.VMEM((R + 1, D), jnp.float32),
            pltpu.SemaphoreType.DMA,
            pltpu.SemaphoreType.DMA,
            pltpu.SemaphoreType.DMA,
        ],
    )(_sc_body)
    return f(x_user, uu_src, uu_dst, ui_src, ui_dst)


def _linear_body(x_ref, agg_ref, w_ref, b_ref, o_ref):
    agg = agg_ref[...]
    agg = jnp.where(jnp.isfinite(agg), agg, 0.0)
    o_ref[...] = (jnp.dot(x_ref[...] + agg, w_ref[...],
                          preferred_element_type=jnp.float32)
                  + b_ref[...])


def _gin_linear(x, agg, W, b):
    blk = 1000
    grid = (N_NODES // blk,)
    return pl.pallas_call(
        _linear_body,
        grid=grid,
        in_specs=[
            pl.BlockSpec((blk, D), lambda i: (i, 0)),
            pl.BlockSpec((blk, D), lambda i: (i, 0)),
            pl.BlockSpec((D, D), lambda i: (0, 0)),
            pl.BlockSpec((1, D), lambda i: (0, 0)),
        ],
        out_specs=pl.BlockSpec((blk, D), lambda i: (i, 0)),
        out_shape=jax.ShapeDtypeStruct((N_NODES, D), jnp.float32),
    )(x, agg, W, b.reshape(1, D))


def kernel(x_user, x_item, edge_index_uu, edge_index_ui,
           W_follows, b_follows, W_clicks, b_clicks):
    agg_uu, agg_ui = _segmax_both(
        x_user,
        edge_index_uu[0], edge_index_uu[1],
        edge_index_ui[0], edge_index_ui[1],
    )
    h_user = _gin_linear(x_user, agg_uu[:N_NODES], W_follows, b_follows)
    h_item = _gin_linear(x_item, agg_ui[:N_NODES], W_clicks, b_clicks)
    return (h_user, h_item)


# speedup vs baseline: 7.3377x; 1.0213x over previous
import functools

import jax
import jax.numpy as jnp
from jax import lax
from jax.experimental import pallas as pl
from jax.experimental.pallas import tpu as pltpu
from jax.experimental.pallas import tpu_sc as plsc

N_NODES = 10000
D = 128
E = 160000

NSUB = 16
R = 632
NPAD = NSUB * R
DUMP = R
C = 4000
NCHUNK = E // C
G = 128


def _segmax_worker(src_hbm, dst_hbm, x_hbm, out_hbm, sub,
                   src_v, dst_v, src_b, dst_b, csrc_v, cdst_v,
                   gidx_v, gdl_v, rows_v, acc_v, sem, csem_a, csem_b):
    lo = sub * R
    iota = lax.iota(jnp.int32, 16)
    minf = jnp.full((16,), -jnp.inf, dtype=jnp.float32)

    def init_row(i, _):
        for j in range(D // 16):
            acc_v[i, pl.ds(j * 16, 16)] = minf
        return 0
    lax.fori_loop(0, R + 1, init_row, 0)

    def accumulate():
        def grp_body(g, _):
            dlv = gdl_v[pl.ds(g * 16, 16)]
            for k in range(16):
                dl = dlv[k]
                e = g * 16 + k
                msgs = [rows_v[e, pl.ds(j * 16, 16)]
                        for j in range(D // 16)]
                for j in range(D // 16):
                    cs = pl.ds(j * 16, 16)
                    acc_v[dl, cs] = jnp.maximum(acc_v[dl, cs], msgs[j])
            return 0
        lax.fori_loop(0, G // 16, grp_body, 0)

    def flush_now():
        pltpu.make_async_copy(x_hbm.at[gidx_v], rows_v, sem).wait()
        accumulate()

    def issue(b):
        for i in range(G // 16):
            gidx_v[pl.ds(i * 16, 16)] = csrc_v[pl.ds(b * G + i * 16, 16)]
            gdl_v[pl.ds(i * 16, 16)] = cdst_v[pl.ds(b * G + i * 16, 16)]
        pltpu.async_copy(x_hbm.at[gidx_v], rows_v, sem)

    def issue_chunk(ch, sbuf, dbuf, csem):
        pltpu.async_copy(src_hbm.at[pl.ds(ch * C, C)], sbuf, csem)
        pltpu.async_copy(dst_hbm.at[pl.ds(ch * C, C)], dbuf, csem)

    def wait_chunk(ch, sbuf, dbuf, csem):
        pltpu.make_async_copy(src_hbm.at[pl.ds(ch * C, C)], sbuf,
                              csem).wait()
        pltpu.make_async_copy(dst_hbm.at[pl.ds(ch * C, C)], dbuf,
                              csem).wait()

    def process_chunk(sbuf, dbuf, carry):
        n, p = carry

        def scan_body(g, n):
            d = dbuf[pl.ds(g * 16, 16)]
            s = sbuf[pl.ds(g * 16, 16)]
            dl = d - lo
            m = (dl >= 0) & (dl < R)
            mi = m.astype(jnp.int32)
            c = plsc.cumsum(mi)
            pos = n + c - 1
            plsc.store_scatter(csrc_v, [pos], s, mask=m)
            plsc.store_scatter(cdst_v, [pos], dl, mask=m)
            return n + c[15]
        n = lax.fori_loop(0, C // 16, scan_body, n)

        nfull = n // G

        def bloop(b, p):
            @pl.when(p == 1)
            def _():
                flush_now()
            issue(b)
            return jnp.int32(1)
        p = lax.fori_loop(0, nfull, bloop, p)

        for i in range(G // 16):
            sv = csrc_v[pl.ds(nfull * G + i * 16, 16)]
            dv = cdst_v[pl.ds(nfull * G + i * 16, 16)]
            csrc_v[pl.ds(i * 16, 16)] = sv
            cdst_v[pl.ds(i * 16, 16)] = dv
        return (n - nfull * G, p)

    issue_chunk(0, src_v, dst_v, csem_a)

    def super_body(i, carry):
        wait_chunk(2 * i, src_v, dst_v, csem_a)
        issue_chunk(2 * i + 1, src_b, dst_b, csem_b)
        carry = process_chunk(src_v, dst_v, carry)

        @pl.when(i < NCHUNK // 2 - 1)
        def _():
            issue_chunk(2 * i + 2, src_v, dst_v, csem_a)
        wait_chunk(2 * i + 1, src_b, dst_b, csem_b)
        carry = process_chunk(src_b, dst_b, carry)
        return carry

    n, p = lax.fori_loop(0, NCHUNK // 2, super_body,
                         (jnp.int32(0), jnp.int32(0)))

    @pl.when(p == 1)
    def _():
        flush_now()

    for i in range(G // 16):
        csrc_v[pl.ds(n + i * 16, 16)] = iota + (i * 16)
        cdst_v[pl.ds(n + i * 16, 16)] = jnp.full((16,), DUMP, jnp.int32)
    nb = (n + G - 1) // G

    def tail_body(b, _):
        issue(b)
        flush_now()
        return 0
    lax.fori_loop(0, nb, tail_body, 0)

    pltpu.sync_copy(acc_v.at[pl.ds(0, R)], out_hbm.at[pl.ds(lo, R)])


def _sc_body(x_hbm, uu_src, uu_dst, ui_src, ui_dst, agg_uu, agg_ui,
             src_v, dst_v, src_b, dst_b, csrc_v, cdst_v, gidx_v, gdl_v,
             rows_v, acc_v, sem, csem_a, csem_b):
    core = lax.axis_index("c")
    sub = lax.axis_index("s")

    @pl.when(core == 0)
    def _():
        _segmax_worker(uu_src, uu_dst, x_hbm, agg_uu, sub,
                       src_v, dst_v, src_b, dst_b, csrc_v, cdst_v,
                       gidx_v, gdl_v, rows_v, acc_v, sem, csem_a, csem_b)

    @pl.when(core == 1)
    def _():
        _segmax_worker(ui_src, ui_dst, x_hbm, agg_ui, sub,
                       src_v, dst_v, src_b, dst_b, csrc_v, cdst_v,
                       gidx_v, gdl_v, rows_v, acc_v, sem, csem_a, csem_b)


def _segmax_both(x_user, uu_src, uu_dst, ui_src, ui_dst):
    mesh = plsc.VectorSubcoreMesh(core_axis_name="c", subcore_axis_name="s")
    f = functools.partial(
        pl.kernel,
        mesh=mesh,
        compiler_params=pltpu.CompilerParams(needs_layout_passes=False),
        out_type=[
            jax.ShapeDtypeStruct((NPAD, D), jnp.float32),
            jax.ShapeDtypeStruct((NPAD, D), jnp.float32),
        ],
        scratch_types=[
            pltpu.VMEM((C,), jnp.int32),
            pltpu.VMEM((C,), jnp.int32),
            pltpu.VMEM((C,), jnp.int32),
            pltpu.VMEM((C,), jnp.int32),
            pltpu.VMEM((C + G,), jnp.int32),
            pltpu.VMEM((C + G,), jnp.int32),
            pltpu.VMEM((G,), jnp.int32),
            pltpu.VMEM((G,), jnp.int32),
            pltpu.VMEM((G, D), jnp.float32),
            pltpu.VMEM((R + 1, D), jnp.float32),
            pltpu.SemaphoreType.DMA,
            pltpu.SemaphoreType.DMA,
            pltpu.SemaphoreType.DMA,
        ],
    )(_sc_body)
    return f(x_user, uu_src, uu_dst, ui_src, ui_dst)


def _linear_body(x_ref, agg_ref, w_ref, b_ref, o_ref):
    agg = agg_ref[...]
    agg = jnp.where(jnp.isfinite(agg), agg, 0.0)
    o_ref[...] = (jnp.dot(x_ref[...] + agg, w_ref[...],
                          preferred_element_type=jnp.float32)
                  + b_ref[...])


def _gin_linear(x, agg, W, b):
    blk = 1000
    grid = (N_NODES // blk,)
    return pl.pallas_call(
        _linear_body,
        grid=grid,
        in_specs=[
            pl.BlockSpec((blk, D), lambda i: (i, 0)),
            pl.BlockSpec((blk, D), lambda i: (i, 0)),
            pl.BlockSpec((D, D), lambda i: (0, 0)),
            pl.BlockSpec((1, D), lambda i: (0, 0)),
        ],
        out_specs=pl.BlockSpec((blk, D), lambda i: (i, 0)),
        out_shape=jax.ShapeDtypeStruct((N_NODES, D), jnp.float32),
    )(x, agg, W, b.reshape(1, D))


def kernel(x_user, x_item, edge_index_uu, edge_index_ui,
           W_follows, b_follows, W_clicks, b_clicks):
    agg_uu, agg_ui = _segmax_both(
        x_user,
        edge_index_uu[0], edge_index_uu[1],
        edge_index_ui[0], edge_index_ui[1],
    )
    h_user = _gin_linear(x_user, agg_uu, W_follows, b_follows)
    h_item = _gin_linear(x_item, agg_ui, W_clicks, b_clicks)
    return (h_user, h_item)


# speedup vs baseline: 8.4554x; 1.1523x over previous
import functools

import jax
import jax.numpy as jnp
from jax import lax
from jax.experimental import pallas as pl
from jax.experimental.pallas import tpu as pltpu
from jax.experimental.pallas import tpu_sc as plsc

N_NODES = 10000
D = 128
E = 160000

NSUB = 16
R = 632
NPAD = NSUB * R
DUMP = R
C = 4000
NCHUNK = E // C
G = 128


def _segmax_worker(src_hbm, dst_hbm, x_hbm, out_hbm, sub,
                   src_v, dst_v, src_b, dst_b, csrc_v, cdst_v,
                   gidx_v, gdl_v, rows_v, acc_v, sem, csem_a, csem_b):
    lo = sub * R
    iota = lax.iota(jnp.int32, 16)
    minf = jnp.full((16,), -jnp.inf, dtype=jnp.float32)

    def init_row(i, _):
        for j in range(D // 16):
            acc_v[i, pl.ds(j * 16, 16)] = minf
        return 0
    lax.fori_loop(0, R + 1, init_row, 0)

    def accumulate():
        def grp_body(g, _):
            dlv = gdl_v[pl.ds(g * 16, 16)]
            for k in range(16):
                dl = dlv[k]
                e = g * 16 + k
                msgs = [rows_v[e, pl.ds(j * 16, 16)]
                        for j in range(D // 16)]
                for j in range(D // 16):
                    cs = pl.ds(j * 16, 16)
                    acc_v[dl, cs] = jnp.maximum(acc_v[dl, cs], msgs[j])
            return 0
        lax.fori_loop(0, G // 16, grp_body, 0)

    def flush_now():
        pltpu.make_async_copy(x_hbm.at[gidx_v], rows_v, sem).wait()
        accumulate()

    def issue(b):
        for i in range(G // 16):
            gidx_v[pl.ds(i * 16, 16)] = csrc_v[pl.ds(b * G + i * 16, 16)]
            gdl_v[pl.ds(i * 16, 16)] = cdst_v[pl.ds(b * G + i * 16, 16)]
        pltpu.async_copy(x_hbm.at[gidx_v], rows_v, sem)

    def issue_chunk(ch, sbuf, dbuf, csem):
        pltpu.async_copy(src_hbm.at[pl.ds(ch * C, C)], sbuf, csem)
        pltpu.async_copy(dst_hbm.at[pl.ds(ch * C, C)], dbuf, csem)

    def wait_chunk(ch, sbuf, dbuf, csem):
        pltpu.make_async_copy(src_hbm.at[pl.ds(ch * C, C)], sbuf,
                              csem).wait()
        pltpu.make_async_copy(dst_hbm.at[pl.ds(ch * C, C)], dbuf,
                              csem).wait()

    def process_chunk(sbuf, dbuf, carry):
        n, p = carry

        def scan_body(g, n):
            d1 = dbuf[pl.ds(g * 32, 16)]
            d2 = dbuf[pl.ds(g * 32 + 16, 16)]
            s1 = sbuf[pl.ds(g * 32, 16)]
            s2 = sbuf[pl.ds(g * 32 + 16, 16)]
            dl1 = d1 - lo
            dl2 = d2 - lo
            m1 = (dl1 >= 0) & (dl1 < R)
            m2 = (dl2 >= 0) & (dl2 < R)
            c1 = plsc.cumsum(m1.astype(jnp.int32))
            c2 = plsc.cumsum(m2.astype(jnp.int32))
            pos1 = n + c1 - 1
            pos2 = pos1[15] + c2
            plsc.store_scatter(csrc_v, [pos1], s1, mask=m1)
            plsc.store_scatter(cdst_v, [pos1], dl1, mask=m1)
            plsc.store_scatter(csrc_v, [pos2], s2, mask=m2)
            plsc.store_scatter(cdst_v, [pos2], dl2, mask=m2)
            return pos2[15] + 1
        n = lax.fori_loop(0, C // 32, scan_body, n)

        nfull = n // G

        def bloop(b, p):
            @pl.when(p == 1)
            def _():
                flush_now()
            issue(b)
            return jnp.int32(1)
        p = lax.fori_loop(0, nfull, bloop, p)

        for i in range(G // 16):
            sv = csrc_v[pl.ds(nfull * G + i * 16, 16)]
            dv = cdst_v[pl.ds(nfull * G + i * 16, 16)]
            csrc_v[pl.ds(i * 16, 16)] = sv
            cdst_v[pl.ds(i * 16, 16)] = dv
        return (n - nfull * G, p)

    issue_chunk(0, src_v, dst_v, csem_a)

    def super_body(i, carry):
        wait_chunk(2 * i, src_v, dst_v, csem_a)
        issue_chunk(2 * i + 1, src_b, dst_b, csem_b)
        carry = process_chunk(src_v, dst_v, carry)

        @pl.when(i < NCHUNK // 2 - 1)
        def _():
            issue_chunk(2 * i + 2, src_v, dst_v, csem_a)
        wait_chunk(2 * i + 1, src_b, dst_b, csem_b)
        carry = process_chunk(src_b, dst_b, carry)
        return carry

    n, p = lax.fori_loop(0, NCHUNK // 2, super_body,
                         (jnp.int32(0), jnp.int32(0)))

    @pl.when(p == 1)
    def _():
        flush_now()

    for i in range(G // 16):
        csrc_v[pl.ds(n + i * 16, 16)] = iota + (i * 16)
        cdst_v[pl.ds(n + i * 16, 16)] = jnp.full((16,), DUMP, jnp.int32)
    nb = (n + G - 1) // G

    def tail_body(b, _):
        issue(b)
        flush_now()
        return 0
    lax.fori_loop(0, nb, tail_body, 0)

    pltpu.sync_copy(acc_v.at[pl.ds(0, R)], out_hbm.at[pl.ds(lo, R)])


def _sc_body(x_hbm, uu_src, uu_dst, ui_src, ui_dst, agg_uu, agg_ui,
             src_v, dst_v, src_b, dst_b, csrc_v, cdst_v, gidx_v, gdl_v,
             rows_v, acc_v, sem, csem_a, csem_b):
    core = lax.axis_index("c")
    sub = lax.axis_index("s")

    @pl.when(core == 0)
    def _():
        _segmax_worker(uu_src, uu_dst, x_hbm, agg_uu, sub,
                       src_v, dst_v, src_b, dst_b, csrc_v, cdst_v,
                       gidx_v, gdl_v, rows_v, acc_v, sem, csem_a, csem_b)

    @pl.when(core == 1)
    def _():
        _segmax_worker(ui_src, ui_dst, x_hbm, agg_ui, sub,
                       src_v, dst_v, src_b, dst_b, csrc_v, cdst_v,
                       gidx_v, gdl_v, rows_v, acc_v, sem, csem_a, csem_b)


def _segmax_both(x_user, uu_src, uu_dst, ui_src, ui_dst):
    mesh = plsc.VectorSubcoreMesh(core_axis_name="c", subcore_axis_name="s")
    f = functools.partial(
        pl.kernel,
        mesh=mesh,
        compiler_params=pltpu.CompilerParams(needs_layout_passes=False),
        out_type=[
            jax.ShapeDtypeStruct((NPAD, D), jnp.float32),
            jax.ShapeDtypeStruct((NPAD, D), jnp.float32),
        ],
        scratch_types=[
            pltpu.VMEM((C,), jnp.int32),
            pltpu.VMEM((C,), jnp.int32),
            pltpu.VMEM((C,), jnp.int32),
            pltpu.VMEM((C,), jnp.int32),
            pltpu.VMEM((C + G,), jnp.int32),
            pltpu.VMEM((C + G,), jnp.int32),
            pltpu.VMEM((G,), jnp.int32),
            pltpu.VMEM((G,), jnp.int32),
            pltpu.VMEM((G, D), jnp.float32),
            pltpu.VMEM((R + 1, D), jnp.float32),
            pltpu.SemaphoreType.DMA,
            pltpu.SemaphoreType.DMA,
            pltpu.SemaphoreType.DMA,
        ],
    )(_sc_body)
    return f(x_user, uu_src, uu_dst, ui_src, ui_dst)


def _linear_body(x_ref, agg_ref, w_ref, b_ref, o_ref):
    agg = agg_ref[...]
    agg = jnp.where(jnp.isfinite(agg), agg, 0.0)
    o_ref[...] = (jnp.dot(x_ref[...] + agg, w_ref[...],
                          preferred_element_type=jnp.float32)
                  + b_ref[...])


def _gin_linear(x, agg, W, b):
    blk = 1000
    grid = (N_NODES // blk,)
    return pl.pallas_call(
        _linear_body,
        grid=grid,
        in_specs=[
            pl.BlockSpec((blk, D), lambda i: (i, 0)),
            pl.BlockSpec((blk, D), lambda i: (i, 0)),
            pl.BlockSpec((D, D), lambda i: (0, 0)),
            pl.BlockSpec((1, D), lambda i: (0, 0)),
        ],
        out_specs=pl.BlockSpec((blk, D), lambda i: (i, 0)),
        out_shape=jax.ShapeDtypeStruct((N_NODES, D), jnp.float32),
    )(x, agg, W, b.reshape(1, D))


def kernel(x_user, x_item, edge_index_uu, edge_index_ui,
           W_follows, b_follows, W_clicks, b_clicks):
    agg_uu, agg_ui = _segmax_both(
        x_user,
        edge_index_uu[0], edge_index_uu[1],
        edge_index_ui[0], edge_index_ui[1],
    )
    h_user = _gin_linear(x_user, agg_uu, W_follows, b_follows)
    h_item = _gin_linear(x_item, agg_ui, W_clicks, b_clicks)
    return (h_user, h_item)
